# baseline (speedup 1.0000x reference)
import jax
import jax.numpy as jnp
from jax import lax
from jax.experimental import pallas as pl
from jax.experimental.pallas import tpu as pltpu
from jax.experimental.pallas import tpu_sc as plsc

N = 10000
E = 320000
D = 128

NC = 2
NS = 16
NW = NC * NS
CHUNK = 128
NROW = E // 125
CPW = NROW // NW
NPAD = N + NW
RPT = 624
TAIL_BASE = NS * RPT
TAIL = N - TAIL_BASE
NH = N + 16


def _tile_rows_copy(s, mk_src, mk_dst):
    pltpu.sync_copy(mk_src(s * RPT, RPT), mk_dst(s * RPT, RPT))

    @pl.when(s == NS - 1)
    def _():
        pltpu.sync_copy(mk_src(TAIL_BASE, TAIL), mk_dst(TAIL_BASE, TAIL))


_SC_MESH = plsc.VectorSubcoreMesh(core_axis_name="c", subcore_axis_name="s")


ND = 10240
SPT = ND // NS


def _deg_body(dst_hbm, out_hbm, idx_v, ones_v, zbuf, deg_sh):
    c = lax.axis_index("c")
    s = lax.axis_index("s")
    wid = c * NS + s
    pltpu.sync_copy(dst_hbm.at[pl.ds(wid * CPW, CPW)], idx_v)

    zero16 = jnp.zeros((16,), jnp.float32)
    one16 = jnp.ones((16,), jnp.float32)
    for k in range(CHUNK // 16):
        ones_v[pl.ds(k * 16, 16)] = one16

    def zb(i, carry):
        zbuf[pl.ds(i * 16, 16)] = zero16
        return carry

    lax.fori_loop(0, SPT // 16, zb, 0)
    pltpu.sync_copy(zbuf, deg_sh.at[pl.ds(s * SPT, SPT)])
    plsc.subcore_barrier()

    def body(j, carry):
        pltpu.sync_copy(ones_v, deg_sh.at[idx_v.at[j]], add=True)
        return carry

    lax.fori_loop(0, CPW, body, 0)
    plsc.subcore_barrier()
    pltpu.sync_copy(deg_sh.at[pl.ds(s * SPT, SPT)],
                    out_hbm.at[pl.ds(c * ND + s * SPT, SPT)])


_deg_kernel = pl.kernel(
    _deg_body,
    out_type=jax.ShapeDtypeStruct((NC * ND,), jnp.float32),
    mesh=_SC_MESH,
    scratch_types=[
        pltpu.VMEM((CPW, CHUNK), jnp.int32),
        pltpu.VMEM((CHUNK,), jnp.float32),
        pltpu.VMEM((SPT,), jnp.float32),
        pltpu.VMEM_SHARED((ND,), jnp.float32),
    ],
)


SB = 16
NSB = CPW // SB


def _agg_body(h_hbm, src_hbm, dst_hbm, zeros_hbm, out_hbm,
              sbig, idst_v, buf_a, acc_sh, sem_a):
    c = lax.axis_index("c")
    s = lax.axis_index("s")
    wid = c * NS + s
    pltpu.sync_copy(dst_hbm.at[pl.ds(wid * CPW, CPW)], idst_v)
    _tile_rows_copy(s, lambda b, n: zeros_hbm.at[pl.ds(b, n)],
                    lambda b, n: acc_sh.at[pl.ds(b, n)])
    plsc.subcore_barrier()

    def super_body(sb, carry):
        pltpu.sync_copy(src_hbm.at[pl.ds(wid * CPW + SB * sb, SB)], sbig)

        for k in range(SB):
            j = SB * sb + k
            pltpu.async_copy(h_hbm.at[sbig.at[k]], buf_a, sem_a).wait()
            pltpu.sync_copy(buf_a, acc_sh.at[idst_v.at[j]], add=True)
        return carry

    lax.fori_loop(0, NSB, super_body, 0)
    plsc.subcore_barrier()
    _tile_rows_copy(s, lambda b, n: acc_sh.at[pl.ds(b, n)],
                    lambda b, n: out_hbm.at[c, pl.ds(b, n)])


_agg_kernel = pl.kernel(
    _agg_body,
    out_type=jax.ShapeDtypeStruct((NC, N, D), jnp.float32),
    mesh=_SC_MESH,
    scratch_types=[
        pltpu.VMEM((SB, CHUNK), jnp.int32),
        pltpu.VMEM((CPW, CHUNK), jnp.int32),
        pltpu.VMEM((CHUNK, D), jnp.float32),
        pltpu.VMEM_SHARED((NPAD, D), jnp.float32),
        pltpu.SemaphoreType.DMA,
    ],
)


RB = 400
GRID = N // RB


def _tc1_body(x_ref, w_ref, dp_ref, hp_ref, dis_ref):
    dis = lax.rsqrt(dp_ref[0] + dp_ref[1] + 1.0)
    h = jnp.dot(x_ref[...], w_ref[...], preferred_element_type=jnp.float32)
    hp_ref[...] = h * dis
    dis_ref[...] = dis


_tc1 = pl.pallas_call(
    _tc1_body,
    grid=(GRID,),
    in_specs=[
        pl.BlockSpec((RB, D), lambda i: (i, 0)),
        pl.BlockSpec((D, D), lambda i: (0, 0)),
        pl.BlockSpec((NC, RB, 1), lambda i: (0, i, 0)),
    ],
    out_specs=[
        pl.BlockSpec((RB, D), lambda i: (i, 0)),
        pl.BlockSpec((RB, 1), lambda i: (i, 0)),
    ],
    out_shape=[
        jax.ShapeDtypeStruct((N, D), jnp.float32),
        jax.ShapeDtypeStruct((N, 1), jnp.float32),
    ],
)


def _tc2_body(p_ref, hp1_ref, dis_ref, b1_ref, w2_ref, hp2_ref):
    ssum = p_ref[0] + p_ref[1] + hp1_ref[...]
    h1 = jnp.maximum(ssum * dis_ref[...] + b1_ref[...], 0.0)
    h2 = jnp.dot(h1, w2_ref[...], preferred_element_type=jnp.float32)
    hp2_ref[...] = h2 * dis_ref[...]


_tc2 = pl.pallas_call(
    _tc2_body,
    grid=(GRID,),
    in_specs=[
        pl.BlockSpec((NC, RB, D), lambda i: (0, i, 0)),
        pl.BlockSpec((RB, D), lambda i: (i, 0)),
        pl.BlockSpec((RB, 1), lambda i: (i, 0)),
        pl.BlockSpec((1, D), lambda i: (0, 0)),
        pl.BlockSpec((D, D), lambda i: (0, 0)),
    ],
    out_specs=pl.BlockSpec((RB, D), lambda i: (i, 0)),
    out_shape=jax.ShapeDtypeStruct((N, D), jnp.float32),
)


def _tc3_body(p_ref, hp2_ref, dis_ref, b2_ref, out_ref):
    ssum = p_ref[0] + p_ref[1] + hp2_ref[...]
    out_ref[...] = ssum * dis_ref[...] + b2_ref[...]


_tc3 = pl.pallas_call(
    _tc3_body,
    grid=(GRID,),
    in_specs=[
        pl.BlockSpec((NC, RB, D), lambda i: (0, i, 0)),
        pl.BlockSpec((RB, D), lambda i: (i, 0)),
        pl.BlockSpec((RB, 1), lambda i: (i, 0)),
        pl.BlockSpec((1, D), lambda i: (0, 0)),
    ],
    out_specs=pl.BlockSpec((RB, D), lambda i: (i, 0)),
    out_shape=jax.ShapeDtypeStruct((N, D), jnp.float32),
)


def _pad_chunks(idx, fill):
    idx2 = idx.reshape(NROW, 125)
    pad = jnp.broadcast_to(fill[:, None] if fill.ndim else fill,
                           (NROW, CHUNK - 125)).astype(jnp.int32)
    return jnp.concatenate([idx2, pad], axis=1)


@jax.jit
def kernel(x, edge_index, W1, b1, W2, b2):
    srcp = _pad_chunks(edge_index[0], jnp.int32(0))
    dumps = N + jnp.arange(NROW, dtype=jnp.int32) // CPW
    dstp = _pad_chunks(edge_index[1], dumps)
    zerosD = jnp.zeros((N, D), jnp.float32)

    deg = _deg_kernel(dstp)
    deg_parts = deg.reshape(NC, ND)[:, :N].reshape(NC, N, 1)
    hp1, dis = _tc1(x, W1, deg_parts)
    parts1 = _agg_kernel(hp1, srcp, dstp, zerosD)
    hp2 = _tc2(parts1, hp1, dis, b1.reshape(1, D), W2)
    parts2 = _agg_kernel(hp2, srcp, dstp, zerosD)
    out = _tc3(parts2, hp2, dis, b2.reshape(1, D))
    return out


# speedup vs baseline: 1.8715x; 1.8715x over previous
import jax
import jax.numpy as jnp
from jax import lax
from jax.experimental import pallas as pl
from jax.experimental.pallas import tpu as pltpu
from jax.experimental.pallas import tpu_sc as plsc

N = 10000
E = 320000
D = 128

NC = 2
NS = 16
NW = NC * NS
CHUNK = 128
NROW = E // 125
CPW = NROW // NW
NPAD = N + NW
RPT = 624
TAIL_BASE = NS * RPT
TAIL = N - TAIL_BASE
NH = N + 16


def _tile_rows_copy(s, mk_src, mk_dst):
    pltpu.sync_copy(mk_src(s * RPT, RPT), mk_dst(s * RPT, RPT))

    @pl.when(s == NS - 1)
    def _():
        pltpu.sync_copy(mk_src(TAIL_BASE, TAIL), mk_dst(TAIL_BASE, TAIL))


_SC_MESH = plsc.VectorSubcoreMesh(core_axis_name="c", subcore_axis_name="s")


ND = 10240
SPT = ND // NS


def _deg_body(dst_hbm, out_hbm, idx_v, ones_v, zbuf, deg_sh):
    c = lax.axis_index("c")
    s = lax.axis_index("s")
    wid = c * NS + s
    pltpu.sync_copy(dst_hbm.at[pl.ds(wid * CPW, CPW)], idx_v)

    zero16 = jnp.zeros((16,), jnp.float32)
    one16 = jnp.ones((16,), jnp.float32)
    for k in range(CHUNK // 16):
        ones_v[pl.ds(k * 16, 16)] = one16

    def zb(i, carry):
        zbuf[pl.ds(i * 16, 16)] = zero16
        return carry

    lax.fori_loop(0, SPT // 16, zb, 0)
    pltpu.sync_copy(zbuf, deg_sh.at[pl.ds(s * SPT, SPT)])
    plsc.subcore_barrier()

    def body(j, carry):
        pltpu.sync_copy(ones_v, deg_sh.at[idx_v.at[j]], add=True)
        return carry

    lax.fori_loop(0, CPW, body, 0)
    plsc.subcore_barrier()
    pltpu.sync_copy(deg_sh.at[pl.ds(s * SPT, SPT)],
                    out_hbm.at[pl.ds(c * ND + s * SPT, SPT)])


_deg_kernel = pl.kernel(
    _deg_body,
    out_type=jax.ShapeDtypeStruct((NC * ND,), jnp.float32),
    mesh=_SC_MESH,
    scratch_types=[
        pltpu.VMEM((CPW, CHUNK), jnp.int32),
        pltpu.VMEM((CHUNK,), jnp.float32),
        pltpu.VMEM((SPT,), jnp.float32),
        pltpu.VMEM_SHARED((ND,), jnp.float32),
    ],
)


ACH = 125


def _agg_body(h_hbm, src_hbm, dst_hbm, zeros_hbm, out_hbm,
              isrc_v, idst_v, buf_a, acc_sh, sem_a, isem):
    c = lax.axis_index("c")
    s = lax.axis_index("s")
    wid = c * NS + s
    cp_s = pltpu.async_copy(src_hbm.at[pl.ds(wid * CPW, CPW)], isrc_v, isem)
    _tile_rows_copy(s, lambda b, n: zeros_hbm.at[pl.ds(b, n)],
                    lambda b, n: acc_sh.at[pl.ds(b, n)])
    pltpu.sync_copy(dst_hbm.at[pl.ds(wid * CPW, CPW)], idst_v)
    cp_s.wait()
    plsc.subcore_barrier()

    def body(j, carry):
        pltpu.async_copy(h_hbm.at[isrc_v.at[j]], buf_a, sem_a).wait()
        pltpu.sync_copy(buf_a, acc_sh.at[idst_v.at[j]], add=True)
        return carry

    lax.fori_loop(0, CPW, body, 0)
    plsc.subcore_barrier()
    _tile_rows_copy(s, lambda b, n: acc_sh.at[pl.ds(b, n)],
                    lambda b, n: out_hbm.at[c, pl.ds(b, n)])


_agg_kernel = pl.kernel(
    _agg_body,
    out_type=jax.ShapeDtypeStruct((NC, N, D), jnp.float32),
    mesh=_SC_MESH,
    scratch_types=[
        pltpu.VMEM((CPW, ACH), jnp.int32),
        pltpu.VMEM((CPW, ACH), jnp.int32),
        pltpu.VMEM((ACH, D), jnp.float32),
        pltpu.VMEM_SHARED((N, D), jnp.float32),
        pltpu.SemaphoreType.DMA,
        pltpu.SemaphoreType.DMA,
    ],
)


RB = 400
GRID = N // RB


def _tc1_body(x_ref, w_ref, dp_ref, hp_ref, dis_ref):
    dis = lax.rsqrt(dp_ref[0] + dp_ref[1] + 1.0)
    h = jnp.dot(x_ref[...], w_ref[...], preferred_element_type=jnp.float32)
    hp_ref[...] = h * dis
    dis_ref[...] = dis


_tc1 = pl.pallas_call(
    _tc1_body,
    grid=(GRID,),
    in_specs=[
        pl.BlockSpec((RB, D), lambda i: (i, 0)),
        pl.BlockSpec((D, D), lambda i: (0, 0)),
        pl.BlockSpec((NC, RB, 1), lambda i: (0, i, 0)),
    ],
    out_specs=[
        pl.BlockSpec((RB, D), lambda i: (i, 0)),
        pl.BlockSpec((RB, 1), lambda i: (i, 0)),
    ],
    out_shape=[
        jax.ShapeDtypeStruct((N, D), jnp.float32),
        jax.ShapeDtypeStruct((N, 1), jnp.float32),
    ],
)


def _tc2_body(p_ref, hp1_ref, dis_ref, b1_ref, w2_ref, hp2_ref):
    ssum = p_ref[0] + p_ref[1] + hp1_ref[...]
    h1 = jnp.maximum(ssum * dis_ref[...] + b1_ref[...], 0.0)
    h2 = jnp.dot(h1, w2_ref[...], preferred_element_type=jnp.float32)
    hp2_ref[...] = h2 * dis_ref[...]


_tc2 = pl.pallas_call(
    _tc2_body,
    grid=(GRID,),
    in_specs=[
        pl.BlockSpec((NC, RB, D), lambda i: (0, i, 0)),
        pl.BlockSpec((RB, D), lambda i: (i, 0)),
        pl.BlockSpec((RB, 1), lambda i: (i, 0)),
        pl.BlockSpec((1, D), lambda i: (0, 0)),
        pl.BlockSpec((D, D), lambda i: (0, 0)),
    ],
    out_specs=pl.BlockSpec((RB, D), lambda i: (i, 0)),
    out_shape=jax.ShapeDtypeStruct((N, D), jnp.float32),
)


def _tc3_body(p_ref, hp2_ref, dis_ref, b2_ref, out_ref):
    ssum = p_ref[0] + p_ref[1] + hp2_ref[...]
    out_ref[...] = ssum * dis_ref[...] + b2_ref[...]


_tc3 = pl.pallas_call(
    _tc3_body,
    grid=(GRID,),
    in_specs=[
        pl.BlockSpec((NC, RB, D), lambda i: (0, i, 0)),
        pl.BlockSpec((RB, D), lambda i: (i, 0)),
        pl.BlockSpec((RB, 1), lambda i: (i, 0)),
        pl.BlockSpec((1, D), lambda i: (0, 0)),
    ],
    out_specs=pl.BlockSpec((RB, D), lambda i: (i, 0)),
    out_shape=jax.ShapeDtypeStruct((N, D), jnp.float32),
)


def _pad_chunks(idx, fill):
    idx2 = idx.reshape(NROW, 125)
    pad = jnp.broadcast_to(fill[:, None] if fill.ndim else fill,
                           (NROW, CHUNK - 125)).astype(jnp.int32)
    return jnp.concatenate([idx2, pad], axis=1)


@jax.jit
def kernel(x, edge_index, W1, b1, W2, b2):
    dumps = N + jnp.arange(NROW, dtype=jnp.int32) // CPW
    dstp = _pad_chunks(edge_index[1], dumps)
    src2 = edge_index[0].reshape(NROW, ACH)
    dst2 = edge_index[1].reshape(NROW, ACH)
    zerosD = jnp.zeros((N, D), jnp.float32)

    deg = _deg_kernel(dstp)
    deg_parts = deg.reshape(NC, ND)[:, :N].reshape(NC, N, 1)
    hp1, dis = _tc1(x, W1, deg_parts)
    parts1 = _agg_kernel(hp1, src2, dst2, zerosD)
    hp2 = _tc2(parts1, hp1, dis, b1.reshape(1, D), W2)
    parts2 = _agg_kernel(hp2, src2, dst2, zerosD)
    out = _tc3(parts2, hp2, dis, b2.reshape(1, D))
    return out


# speedup vs baseline: 2.5091x; 1.3407x over previous
import jax
import jax.numpy as jnp
from jax import lax
from jax.experimental import pallas as pl
from jax.experimental.pallas import tpu as pltpu
from jax.experimental.pallas import tpu_sc as plsc

N = 10000
E = 320000
D = 128

NC = 2
NS = 16
NW = NC * NS
CHUNK = 128
NROW = E // 125
CPW = NROW // NW
NPAD = N + NW
RPT = 624
TAIL_BASE = NS * RPT
TAIL = N - TAIL_BASE
NH = N + 16


def _tile_rows_copy(s, mk_src, mk_dst):
    pltpu.sync_copy(mk_src(s * RPT, RPT), mk_dst(s * RPT, RPT))

    @pl.when(s == NS - 1)
    def _():
        pltpu.sync_copy(mk_src(TAIL_BASE, TAIL), mk_dst(TAIL_BASE, TAIL))


_SC_MESH = plsc.VectorSubcoreMesh(core_axis_name="c", subcore_axis_name="s")


ND = 10240
SPT = ND // NS


def _deg_body(dst_hbm, out_hbm, idx_v, ones_v, zbuf, deg_sh):
    c = lax.axis_index("c")
    s = lax.axis_index("s")
    wid = c * NS + s
    pltpu.sync_copy(dst_hbm.at[pl.ds(wid * CPW, CPW)], idx_v)

    zero16 = jnp.zeros((16,), jnp.float32)
    one16 = jnp.ones((16,), jnp.float32)
    for k in range(CHUNK // 16):
        ones_v[pl.ds(k * 16, 16)] = one16

    def zb(i, carry):
        zbuf[pl.ds(i * 16, 16)] = zero16
        return carry

    lax.fori_loop(0, SPT // 16, zb, 0)
    pltpu.sync_copy(zbuf, deg_sh.at[pl.ds(s * SPT, SPT)])
    plsc.subcore_barrier()

    def body(j, carry):
        pltpu.sync_copy(ones_v, deg_sh.at[idx_v.at[j]], add=True)
        return carry

    lax.fori_loop(0, CPW, body, 0)
    plsc.subcore_barrier()
    pltpu.sync_copy(deg_sh.at[pl.ds(s * SPT, SPT)],
                    out_hbm.at[pl.ds(c * ND + s * SPT, SPT)])


_deg_kernel = pl.kernel(
    _deg_body,
    out_type=jax.ShapeDtypeStruct((NC * ND,), jnp.float32),
    mesh=_SC_MESH,
    scratch_types=[
        pltpu.VMEM((CPW, CHUNK), jnp.int32),
        pltpu.VMEM((CHUNK,), jnp.float32),
        pltpu.VMEM((SPT,), jnp.float32),
        pltpu.VMEM_SHARED((ND,), jnp.float32),
    ],
)


ACH = 125


SB = 16
NSB = CPW // SB


def _agg_body(h_hbm, src_hbm, dst_hbm, zeros_hbm, out_hbm,
              sbig, idst_v, buf_a, buf_b, acc_sh, sem_a, sem_b):
    c = lax.axis_index("c")
    s = lax.axis_index("s")
    wid = c * NS + s
    pltpu.sync_copy(dst_hbm.at[pl.ds(wid * CPW, CPW)], idst_v)
    _tile_rows_copy(s, lambda b, n: zeros_hbm.at[pl.ds(b, n)],
                    lambda b, n: acc_sh.at[pl.ds(b, n)])
    plsc.subcore_barrier()

    def super_body(sb, carry):
        pltpu.sync_copy(src_hbm.at[pl.ds(wid * CPW + SB * sb, SB)], sbig)

        descs = [None] * SB
        descs[0] = pltpu.async_copy(h_hbm.at[sbig.at[0]], buf_a, sem_a)
        for k in range(SB):
            j = SB * sb + k
            cur = buf_a if k % 2 == 0 else buf_b
            nxt, nsem = (buf_b, sem_b) if k % 2 == 0 else (buf_a, sem_a)
            if k < SB - 1:
                descs[k + 1] = pltpu.async_copy(
                    h_hbm.at[sbig.at[k + 1]], nxt, nsem)
            descs[k].wait()
            pltpu.sync_copy(cur, acc_sh.at[idst_v.at[j]], add=True)
        return carry

    lax.fori_loop(0, NSB, super_body, 0)
    plsc.subcore_barrier()
    _tile_rows_copy(s, lambda b, n: acc_sh.at[pl.ds(b, n)],
                    lambda b, n: out_hbm.at[c, pl.ds(b, n)])


_agg_kernel = pl.kernel(
    _agg_body,
    out_type=jax.ShapeDtypeStruct((NC, N, D), jnp.float32),
    mesh=_SC_MESH,
    scratch_types=[
        pltpu.VMEM((SB, ACH), jnp.int32),
        pltpu.VMEM((CPW, ACH), jnp.int32),
        pltpu.VMEM((ACH, D), jnp.float32),
        pltpu.VMEM((ACH, D), jnp.float32),
        pltpu.VMEM_SHARED((N, D), jnp.float32),
        pltpu.SemaphoreType.DMA,
        pltpu.SemaphoreType.DMA,
    ],
)


RB = 400
GRID = N // RB


def _tc1_body(x_ref, w_ref, dp_ref, hp_ref, dis_ref):
    dis = lax.rsqrt(dp_ref[0] + dp_ref[1] + 1.0)
    h = jnp.dot(x_ref[...], w_ref[...], preferred_element_type=jnp.float32)
    hp_ref[...] = h * dis
    dis_ref[...] = dis


_tc1 = pl.pallas_call(
    _tc1_body,
    grid=(GRID,),
    in_specs=[
        pl.BlockSpec((RB, D), lambda i: (i, 0)),
        pl.BlockSpec((D, D), lambda i: (0, 0)),
        pl.BlockSpec((NC, RB, 1), lambda i: (0, i, 0)),
    ],
    out_specs=[
        pl.BlockSpec((RB, D), lambda i: (i, 0)),
        pl.BlockSpec((RB, 1), lambda i: (i, 0)),
    ],
    out_shape=[
        jax.ShapeDtypeStruct((N, D), jnp.float32),
        jax.ShapeDtypeStruct((N, 1), jnp.float32),
    ],
)


def _tc2_body(p_ref, hp1_ref, dis_ref, b1_ref, w2_ref, hp2_ref):
    ssum = p_ref[0] + p_ref[1] + hp1_ref[...]
    h1 = jnp.maximum(ssum * dis_ref[...] + b1_ref[...], 0.0)
    h2 = jnp.dot(h1, w2_ref[...], preferred_element_type=jnp.float32)
    hp2_ref[...] = h2 * dis_ref[...]


_tc2 = pl.pallas_call(
    _tc2_body,
    grid=(GRID,),
    in_specs=[
        pl.BlockSpec((NC, RB, D), lambda i: (0, i, 0)),
        pl.BlockSpec((RB, D), lambda i: (i, 0)),
        pl.BlockSpec((RB, 1), lambda i: (i, 0)),
        pl.BlockSpec((1, D), lambda i: (0, 0)),
        pl.BlockSpec((D, D), lambda i: (0, 0)),
    ],
    out_specs=pl.BlockSpec((RB, D), lambda i: (i, 0)),
    out_shape=jax.ShapeDtypeStruct((N, D), jnp.float32),
)


def _tc3_body(p_ref, hp2_ref, dis_ref, b2_ref, out_ref):
    ssum = p_ref[0] + p_ref[1] + hp2_ref[...]
    out_ref[...] = ssum * dis_ref[...] + b2_ref[...]


_tc3 = pl.pallas_call(
    _tc3_body,
    grid=(GRID,),
    in_specs=[
        pl.BlockSpec((NC, RB, D), lambda i: (0, i, 0)),
        pl.BlockSpec((RB, D), lambda i: (i, 0)),
        pl.BlockSpec((RB, 1), lambda i: (i, 0)),
        pl.BlockSpec((1, D), lambda i: (0, 0)),
    ],
    out_specs=pl.BlockSpec((RB, D), lambda i: (i, 0)),
    out_shape=jax.ShapeDtypeStruct((N, D), jnp.float32),
)


def _pad_chunks(idx, fill):
    idx2 = idx.reshape(NROW, 125)
    pad = jnp.broadcast_to(fill[:, None] if fill.ndim else fill,
                           (NROW, CHUNK - 125)).astype(jnp.int32)
    return jnp.concatenate([idx2, pad], axis=1)


@jax.jit
def kernel(x, edge_index, W1, b1, W2, b2):
    dumps = N + jnp.arange(NROW, dtype=jnp.int32) // CPW
    dstp = _pad_chunks(edge_index[1], dumps)
    src2 = edge_index[0].reshape(NROW, ACH)
    dst2 = edge_index[1].reshape(NROW, ACH)
    zerosD = jnp.zeros((N, D), jnp.float32)

    deg = _deg_kernel(dstp)
    deg_parts = deg.reshape(NC, ND)[:, :N].reshape(NC, N, 1)
    hp1, dis = _tc1(x, W1, deg_parts)
    parts1 = _agg_kernel(hp1, src2, dst2, zerosD)
    hp2 = _tc2(parts1, hp1, dis, b1.reshape(1, D), W2)
    parts2 = _agg_kernel(hp2, src2, dst2, zerosD)
    out = _tc3(parts2, hp2, dis, b2.reshape(1, D))
    return out
